# Initial kernel scaffold; baseline (speedup 1.0000x reference)
import jax
import jax.numpy as jnp
from jax.experimental import pallas as pl


def kernel(x1, edge_index1, x2, edge_index2, batch1, batch2, W1_0, b1_0, W1_1, b1_1, W2_0, b2_0, W2_1, b2_1, Wm0, bm0, Wm1, bm1, Wm2, bm2, Wm3, bm3):
    raise NotImplementedError("write your pallas kernel here")


# speedup vs baseline: 14.8225x; 14.8225x over previous
import functools

import jax
import jax.numpy as jnp
from jax import lax
from jax.experimental import pallas as pl
from jax.experimental.pallas import tpu as pltpu
from jax.experimental.pallas import tpu_sc as plsc

N = 10000
E = 320000
D = 128
G = 64
HID = 128

NS = 16
ROWS_PT = 624
TAIL_OFF = NS * ROWS_PT
TAIL = N - TAIL_OFF
K = 80
KJ = 25
IDX_LOAD = K * KJ
N_OUTER = E // NS // IDX_LOAD


def _sc_mesh():
    return plsc.VectorSubcoreMesh(core_axis_name="c", subcore_axis_name="s",
                                  num_cores=2, num_subcores=NS)


def _sc_degree_body(dst_hbm, out_hbm, ones_v, didx_v, acc_sh):
    c = lax.axis_index("c")
    s = lax.axis_index("s")

    def fill(i, _):
        for l in range(D // 16):
            ones_v[i, pl.ds(16 * l, 16)] = jnp.ones((16,), jnp.float32)
        return 0

    lax.fori_loop(0, K, fill, 0)
    for i in range(7):
        pltpu.sync_copy(ones_v,
                        acc_sh.at[pl.ds(s * ROWS_PT + i * K, K)])
    pltpu.sync_copy(ones_v.at[pl.ds(0, ROWS_PT - 7 * K)],
                    acc_sh.at[pl.ds(s * ROWS_PT + 7 * K, ROWS_PT - 7 * K)])

    @pl.when(s == NS - 1)
    def _():
        pltpu.sync_copy(ones_v.at[pl.ds(0, TAIL)],
                        acc_sh.at[pl.ds(TAIL_OFF, TAIL)])

    plsc.subcore_barrier()

    def outer(it, _):
        g = s * N_OUTER + it
        pltpu.sync_copy(dst_hbm.at[c, g], didx_v)

        def inner(j, _):
            pltpu.sync_copy(ones_v, acc_sh.at[didx_v.at[j]], add=True)
            return 0

        lax.fori_loop(0, KJ, inner, 0)
        return 0

    lax.fori_loop(0, N_OUTER, outer, 0)
    plsc.subcore_barrier()
    pltpu.sync_copy(acc_sh.at[pl.ds(s * ROWS_PT, ROWS_PT)],
                    out_hbm.at[c, pl.ds(s * ROWS_PT, ROWS_PT)])

    @pl.when(s == NS - 1)
    def _():
        pltpu.sync_copy(acc_sh.at[pl.ds(TAIL_OFF, TAIL)],
                        out_hbm.at[c, pl.ds(TAIL_OFF, TAIL)])


@jax.jit
def _sc_degree(dst_r):
    return pl.kernel(
        _sc_degree_body,
        out_type=jax.ShapeDtypeStruct((2, N, D), jnp.float32),
        mesh=_sc_mesh(),
        scratch_types=[
            pltpu.VMEM((K, D), jnp.float32),
            pltpu.VMEM((KJ, K), jnp.int32),
            pltpu.VMEM_SHARED((N, D), jnp.float32),
        ],
    )(dst_r)


def _sc_prop_body(ys_hbm, src_hbm, dst_hbm, out_hbm,
                  sidx_v, didx_v, rows_v, acc_sh, sem):
    c = lax.axis_index("c")
    s = lax.axis_index("s")

    pltpu.sync_copy(ys_hbm.at[pl.ds(c * N + s * ROWS_PT, ROWS_PT)],
                    acc_sh.at[pl.ds(s * ROWS_PT, ROWS_PT)])

    @pl.when(s == NS - 1)
    def _():
        pltpu.sync_copy(ys_hbm.at[pl.ds(c * N + TAIL_OFF, TAIL)],
                        acc_sh.at[pl.ds(TAIL_OFF, TAIL)])

    plsc.subcore_barrier()

    def outer(it, _):
        g = s * N_OUTER + it
        pltpu.sync_copy(src_hbm.at[c, g], sidx_v)
        pltpu.sync_copy(dst_hbm.at[c, g], didx_v)

        def inner(j, _):
            pltpu.async_copy(ys_hbm.at[sidx_v.at[j]], rows_v, sem).wait()
            pltpu.sync_copy(rows_v, acc_sh.at[didx_v.at[j]], add=True)
            return 0

        lax.fori_loop(0, KJ, inner, 0)
        return 0

    lax.fori_loop(0, N_OUTER, outer, 0)
    plsc.subcore_barrier()
    pltpu.sync_copy(acc_sh.at[pl.ds(s * ROWS_PT, ROWS_PT)],
                    out_hbm.at[c, pl.ds(s * ROWS_PT, ROWS_PT)])

    @pl.when(s == NS - 1)
    def _():
        pltpu.sync_copy(acc_sh.at[pl.ds(TAIL_OFF, TAIL)],
                        out_hbm.at[c, pl.ds(TAIL_OFF, TAIL)])


@jax.jit
def _sc_prop(ys2d, src_r, dst_r):
    return pl.kernel(
        _sc_prop_body,
        out_type=jax.ShapeDtypeStruct((2, N, D), jnp.float32),
        mesh=_sc_mesh(),
        scratch_types=[
            pltpu.VMEM((KJ, K), jnp.int32),
            pltpu.VMEM((KJ, K), jnp.int32),
            pltpu.VMEM((K, D), jnp.float32),
            pltpu.VMEM_SHARED((N, D), jnp.float32),
            pltpu.SemaphoreType.DMA,
        ],
    )(ys2d, src_r, dst_r)


RT = 1000
NRT = N // RT


def _tc1_body(x_ref, w_ref, d_ref, o_ref):
    dis = lax.rsqrt(d_ref[0, :, 0:1])
    o_ref[0] = jnp.dot(x_ref[0], w_ref[0],
                       preferred_element_type=jnp.float32) * dis


@jax.jit
def _tc_first(xs, ws, degp):
    return pl.pallas_call(
        _tc1_body,
        grid=(2, NRT),
        in_specs=[
            pl.BlockSpec((1, RT, D), lambda b, r: (b, r, 0)),
            pl.BlockSpec((1, D, D), lambda b, r: (b, 0, 0)),
            pl.BlockSpec((1, RT, D), lambda b, r: (b, r, 0)),
        ],
        out_specs=pl.BlockSpec((1, RT, D), lambda b, r: (b, r, 0)),
        out_shape=jax.ShapeDtypeStruct((2, N, D), jnp.float32),
    )(xs, ws, degp)


def _tc2_body(p_ref, d_ref, b_ref, w_ref, o_ref):
    dis = lax.rsqrt(d_ref[0, :, 0:1])
    h = jnp.maximum(p_ref[0] * dis + b_ref[0], 0.0)
    o_ref[0] = jnp.dot(h, w_ref[0], preferred_element_type=jnp.float32) * dis


@jax.jit
def _tc_mid(p1, degp, b0s, w1s):
    return pl.pallas_call(
        _tc2_body,
        grid=(2, NRT),
        in_specs=[
            pl.BlockSpec((1, RT, D), lambda b, r: (b, r, 0)),
            pl.BlockSpec((1, RT, D), lambda b, r: (b, r, 0)),
            pl.BlockSpec((1, 1, D), lambda b, r: (b, 0, 0)),
            pl.BlockSpec((1, D, D), lambda b, r: (b, 0, 0)),
        ],
        out_specs=pl.BlockSpec((1, RT, D), lambda b, r: (b, r, 0)),
        out_shape=jax.ShapeDtypeStruct((2, N, D), jnp.float32),
    )(p1, degp, b0s, w1s)


def _tc3_body(p_ref, d_ref, b_ref, bt_ref, wm0_ref, bm0_ref, wm1_ref,
              bm1_ref, wm2_ref, bm2_ref, wm3_ref, bm3_ref, o_ref,
              sums, cnts):
    b = pl.program_id(0)
    r = pl.program_id(1)

    @pl.when((b == 0) & (r == 0))
    def _():
        sums[...] = jnp.zeros((2, G, D), jnp.float32)
        cnts[...] = jnp.zeros((2, G, D), jnp.float32)

    dis = lax.rsqrt(d_ref[0, :, 0:1])
    h = jnp.maximum(p_ref[0] * dis + b_ref[0], 0.0)
    bt = bt_ref[0, :, 0:1]
    iota = lax.broadcasted_iota(jnp.int32, (RT, G), 1).astype(jnp.float32)
    oh = (bt == iota).astype(jnp.float32)
    dn = (((0,), (0,)), ((), ()))
    psum = lax.dot_general(oh, h, dn, preferred_element_type=jnp.float32)
    pcnt = lax.dot_general(oh, jnp.ones((RT, D), jnp.float32), dn,
                           preferred_element_type=jnp.float32)
    sums[pl.ds(b, 1)] += psum[None]
    cnts[pl.ds(b, 1)] += pcnt[None]

    @pl.when((b == 1) & (r == NRT - 1))
    def _():
        m0 = sums[0] / jnp.maximum(cnts[0], 1.0)
        m1 = sums[1] / jnp.maximum(cnts[1], 1.0)
        z = jnp.concatenate([m0, m1], axis=1)
        z = jnp.maximum(jnp.dot(z, wm0_ref[...],
                                preferred_element_type=jnp.float32)
                        + bm0_ref[...], 0.0)
        z = jnp.maximum(jnp.dot(z, wm1_ref[...],
                                preferred_element_type=jnp.float32)
                        + bm1_ref[...], 0.0)
        z = jnp.maximum(jnp.dot(z, wm2_ref[...],
                                preferred_element_type=jnp.float32)
                        + bm2_ref[...], 0.0)
        o_ref[...] = jnp.dot(z, wm3_ref[...],
                             preferred_element_type=jnp.float32) + bm3_ref[...]


@jax.jit
def _tc_final(p2, degp, b1s, batchb, wm0, bm0, wm1, bm1, wm2, bm2,
              wm3p, bm3p):
    fixed = lambda b, r: (0, 0)
    return pl.pallas_call(
        _tc3_body,
        grid=(2, NRT),
        in_specs=[
            pl.BlockSpec((1, RT, D), lambda b, r: (b, r, 0)),
            pl.BlockSpec((1, RT, D), lambda b, r: (b, r, 0)),
            pl.BlockSpec((1, 1, D), lambda b, r: (b, 0, 0)),
            pl.BlockSpec((1, RT, 16), lambda b, r: (b, r, 0)),
            pl.BlockSpec((2 * D, HID), fixed),
            pl.BlockSpec((1, HID), fixed),
            pl.BlockSpec((HID, HID), fixed),
            pl.BlockSpec((1, HID), fixed),
            pl.BlockSpec((HID, HID), fixed),
            pl.BlockSpec((1, HID), fixed),
            pl.BlockSpec((HID, HID), fixed),
            pl.BlockSpec((1, HID), fixed),
        ],
        out_specs=pl.BlockSpec((G, HID), fixed),
        out_shape=jax.ShapeDtypeStruct((G, HID), jnp.float32),
        scratch_shapes=[
            pltpu.VMEM((2, G, D), jnp.float32),
            pltpu.VMEM((2, G, D), jnp.float32),
        ],
    )(p2, degp, b1s, batchb, wm0, bm0, wm1, bm1, wm2, bm2, wm3p, bm3p)


def kernel(x1, edge_index1, x2, edge_index2, batch1, batch2,
           W1_0, b1_0, W1_1, b1_1, W2_0, b2_0, W2_1, b2_1,
           Wm0, bm0, Wm1, bm1, Wm2, bm2, Wm3, bm3):
    src_r = jnp.stack([edge_index1[0], edge_index2[0] + N]).reshape(
        2, NS * N_OUTER, KJ, K)
    dst_r = jnp.stack([edge_index1[1], edge_index2[1]]).reshape(
        2, NS * N_OUTER, KJ, K)
    xs = jnp.stack([x1, x2])
    w0s = jnp.stack([W1_0, W2_0])
    w1s = jnp.stack([W1_1, W2_1])
    b0s = jnp.stack([b1_0, b2_0]).reshape(2, 1, D)
    b1s = jnp.stack([b1_1, b2_1]).reshape(2, 1, D)
    batchb = jnp.broadcast_to(
        jnp.stack([batch1, batch2]).astype(jnp.float32)[:, :, None],
        (2, N, 16))
    wm3p = jnp.pad(Wm3, ((0, 0), (0, HID - Wm3.shape[1])))
    bm3p = jnp.pad(bm3, (0, HID - bm3.shape[0])).reshape(1, HID)
    bm0r = bm0.reshape(1, HID)
    bm1r = bm1.reshape(1, HID)
    bm2r = bm2.reshape(1, HID)

    degp = _sc_degree(dst_r)
    y1 = _tc_first(xs, w0s, degp)
    p1 = _sc_prop(y1.reshape(2 * N, D), src_r, dst_r)
    y2 = _tc_mid(p1, degp, b0s, w1s)
    p2 = _sc_prop(y2.reshape(2 * N, D), src_r, dst_r)
    out = _tc_final(p2, degp, b1s, batchb, Wm0, bm0r, Wm1, bm1r,
                    Wm2, bm2r, wm3p, bm3p)
    return out[:, :Wm3.shape[1]]


# speedup vs baseline: 21.7063x; 1.4644x over previous
import functools

import jax
import jax.numpy as jnp
from jax import lax
from jax.experimental import pallas as pl
from jax.experimental.pallas import tpu as pltpu
from jax.experimental.pallas import tpu_sc as plsc

N = 10000
E = 320000
D = 128
G = 64
HID = 128

NS = 16
ROWS_PT = 624
TAIL_OFF = NS * ROWS_PT
TAIL = N - TAIL_OFF
K = 80
KJ = 25
IDX_LOAD = K * KJ
N_OUTER = E // NS // IDX_LOAD


def _sc_mesh():
    return plsc.VectorSubcoreMesh(core_axis_name="c", subcore_axis_name="s",
                                  num_cores=2, num_subcores=NS)


def _sc_degree_body(dst_hbm, out_hbm, ones_v, didx_v, acc_sh, ssem):
    c = lax.axis_index("c")
    s = lax.axis_index("s")

    def fill(i, _):
        for l in range(D // 16):
            ones_v[i, pl.ds(16 * l, 16)] = jnp.ones((16,), jnp.float32)
        return 0

    lax.fori_loop(0, K, fill, 0)
    for i in range(7):
        pltpu.sync_copy(ones_v,
                        acc_sh.at[pl.ds(s * ROWS_PT + i * K, K)])
    pltpu.sync_copy(ones_v.at[pl.ds(0, ROWS_PT - 7 * K)],
                    acc_sh.at[pl.ds(s * ROWS_PT + 7 * K, ROWS_PT - 7 * K)])

    @pl.when(s == NS - 1)
    def _():
        pltpu.sync_copy(ones_v.at[pl.ds(0, TAIL)],
                        acc_sh.at[pl.ds(TAIL_OFF, TAIL)])

    plsc.subcore_barrier()

    def outer(it, _):
        g = s * N_OUTER + it
        pltpu.sync_copy(dst_hbm.at[c, g], didx_v)
        descs = [pltpu.async_copy(ones_v, acc_sh.at[didx_v.at[j]], ssem,
                                  add=True) for j in range(KJ)]
        for d in descs:
            d.wait()
        return 0

    lax.fori_loop(0, N_OUTER, outer, 0)
    plsc.subcore_barrier()
    pltpu.sync_copy(acc_sh.at[pl.ds(s * ROWS_PT, ROWS_PT)],
                    out_hbm.at[c, pl.ds(s * ROWS_PT, ROWS_PT)])

    @pl.when(s == NS - 1)
    def _():
        pltpu.sync_copy(acc_sh.at[pl.ds(TAIL_OFF, TAIL)],
                        out_hbm.at[c, pl.ds(TAIL_OFF, TAIL)])


@jax.jit
def _sc_degree(dst_r):
    return pl.kernel(
        _sc_degree_body,
        out_type=jax.ShapeDtypeStruct((2, N, D), jnp.float32),
        mesh=_sc_mesh(),
        scratch_types=[
            pltpu.VMEM((K, D), jnp.float32),
            pltpu.VMEM((KJ, K), jnp.int32),
            pltpu.VMEM_SHARED((N, D), jnp.float32),
            pltpu.SemaphoreType.DMA,
        ],
    )(dst_r)


def _sc_prop_body(ys_hbm, src_hbm, dst_hbm, out_hbm,
                  sidx_v, didx_v, rows0_v, rows1_v, acc_sh,
                  gsem0, gsem1, ssem0, ssem1):
    c = lax.axis_index("c")
    s = lax.axis_index("s")

    pltpu.sync_copy(ys_hbm.at[pl.ds(c * N + s * ROWS_PT, ROWS_PT)],
                    acc_sh.at[pl.ds(s * ROWS_PT, ROWS_PT)])

    @pl.when(s == NS - 1)
    def _():
        pltpu.sync_copy(ys_hbm.at[pl.ds(c * N + TAIL_OFF, TAIL)],
                        acc_sh.at[pl.ds(TAIL_OFF, TAIL)])

    plsc.subcore_barrier()

    rows = [rows0_v, rows1_v]
    gsem = [gsem0, gsem1]
    ssem = [ssem0, ssem1]

    def outer(it, _):
        g = s * N_OUTER + it
        pltpu.sync_copy(src_hbm.at[c, g], sidx_v)
        pltpu.sync_copy(dst_hbm.at[c, g], didx_v)
        dg = {}
        ds = {}
        for j in range(KJ):
            p = j % 2
            if j >= 2:
                ds[j - 2].wait()
            dg[j] = pltpu.async_copy(ys_hbm.at[sidx_v.at[j]], rows[p],
                                     gsem[p])
            if j >= 1:
                q = (j - 1) % 2
                dg[j - 1].wait()
                ds[j - 1] = pltpu.async_copy(rows[q],
                                             acc_sh.at[didx_v.at[j - 1]],
                                             ssem[q], add=True)
        pl_last = (KJ - 1) % 2
        dg[KJ - 1].wait()
        ds[KJ - 1] = pltpu.async_copy(rows[pl_last],
                                      acc_sh.at[didx_v.at[KJ - 1]],
                                      ssem[pl_last], add=True)
        ds[KJ - 2].wait()
        ds[KJ - 1].wait()
        return 0

    lax.fori_loop(0, N_OUTER, outer, 0)
    plsc.subcore_barrier()
    pltpu.sync_copy(acc_sh.at[pl.ds(s * ROWS_PT, ROWS_PT)],
                    out_hbm.at[c, pl.ds(s * ROWS_PT, ROWS_PT)])

    @pl.when(s == NS - 1)
    def _():
        pltpu.sync_copy(acc_sh.at[pl.ds(TAIL_OFF, TAIL)],
                        out_hbm.at[c, pl.ds(TAIL_OFF, TAIL)])


@jax.jit
def _sc_prop(ys2d, src_r, dst_r):
    return pl.kernel(
        _sc_prop_body,
        out_type=jax.ShapeDtypeStruct((2, N, D), jnp.float32),
        mesh=_sc_mesh(),
        scratch_types=[
            pltpu.VMEM((KJ, K), jnp.int32),
            pltpu.VMEM((KJ, K), jnp.int32),
            pltpu.VMEM((K, D), jnp.float32),
            pltpu.VMEM((K, D), jnp.float32),
            pltpu.VMEM_SHARED((N, D), jnp.float32),
            pltpu.SemaphoreType.DMA,
            pltpu.SemaphoreType.DMA,
            pltpu.SemaphoreType.DMA,
            pltpu.SemaphoreType.DMA,
        ],
    )(ys2d, src_r, dst_r)


RT = 1000
NRT = N // RT


def _tc1_body(x_ref, w_ref, d_ref, o_ref):
    dis = lax.rsqrt(d_ref[0, :, 0:1])
    o_ref[0] = jnp.dot(x_ref[0], w_ref[0],
                       preferred_element_type=jnp.float32) * dis


@jax.jit
def _tc_first(xs, ws, degp):
    return pl.pallas_call(
        _tc1_body,
        grid=(2, NRT),
        in_specs=[
            pl.BlockSpec((1, RT, D), lambda b, r: (b, r, 0)),
            pl.BlockSpec((1, D, D), lambda b, r: (b, 0, 0)),
            pl.BlockSpec((1, RT, D), lambda b, r: (b, r, 0)),
        ],
        out_specs=pl.BlockSpec((1, RT, D), lambda b, r: (b, r, 0)),
        out_shape=jax.ShapeDtypeStruct((2, N, D), jnp.float32),
    )(xs, ws, degp)


def _tc2_body(p_ref, d_ref, b_ref, w_ref, o_ref):
    dis = lax.rsqrt(d_ref[0, :, 0:1])
    h = jnp.maximum(p_ref[0] * dis + b_ref[0], 0.0)
    o_ref[0] = jnp.dot(h, w_ref[0], preferred_element_type=jnp.float32) * dis


@jax.jit
def _tc_mid(p1, degp, b0s, w1s):
    return pl.pallas_call(
        _tc2_body,
        grid=(2, NRT),
        in_specs=[
            pl.BlockSpec((1, RT, D), lambda b, r: (b, r, 0)),
            pl.BlockSpec((1, RT, D), lambda b, r: (b, r, 0)),
            pl.BlockSpec((1, 1, D), lambda b, r: (b, 0, 0)),
            pl.BlockSpec((1, D, D), lambda b, r: (b, 0, 0)),
        ],
        out_specs=pl.BlockSpec((1, RT, D), lambda b, r: (b, r, 0)),
        out_shape=jax.ShapeDtypeStruct((2, N, D), jnp.float32),
    )(p1, degp, b0s, w1s)


def _tc3_body(p_ref, d_ref, b_ref, bt_ref, wm0_ref, bm0_ref, wm1_ref,
              bm1_ref, wm2_ref, bm2_ref, wm3_ref, bm3_ref, o_ref,
              sums, cnts):
    b = pl.program_id(0)
    r = pl.program_id(1)

    @pl.when((b == 0) & (r == 0))
    def _():
        sums[...] = jnp.zeros((2, G, D), jnp.float32)
        cnts[...] = jnp.zeros((2, G, D), jnp.float32)

    dis = lax.rsqrt(d_ref[0, :, 0:1])
    h = jnp.maximum(p_ref[0] * dis + b_ref[0], 0.0)
    bt = bt_ref[0, :, 0:1]
    iota = lax.broadcasted_iota(jnp.int32, (RT, G), 1).astype(jnp.float32)
    oh = (bt == iota).astype(jnp.float32)
    dn = (((0,), (0,)), ((), ()))
    psum = lax.dot_general(oh, h, dn, preferred_element_type=jnp.float32)
    pcnt = lax.dot_general(oh, jnp.ones((RT, D), jnp.float32), dn,
                           preferred_element_type=jnp.float32)
    sums[pl.ds(b, 1)] += psum[None]
    cnts[pl.ds(b, 1)] += pcnt[None]

    @pl.when((b == 1) & (r == NRT - 1))
    def _():
        m0 = sums[0] / jnp.maximum(cnts[0], 1.0)
        m1 = sums[1] / jnp.maximum(cnts[1], 1.0)
        z = jnp.concatenate([m0, m1], axis=1)
        z = jnp.maximum(jnp.dot(z, wm0_ref[...],
                                preferred_element_type=jnp.float32)
                        + bm0_ref[...], 0.0)
        z = jnp.maximum(jnp.dot(z, wm1_ref[...],
                                preferred_element_type=jnp.float32)
                        + bm1_ref[...], 0.0)
        z = jnp.maximum(jnp.dot(z, wm2_ref[...],
                                preferred_element_type=jnp.float32)
                        + bm2_ref[...], 0.0)
        o_ref[...] = jnp.dot(z, wm3_ref[...],
                             preferred_element_type=jnp.float32) + bm3_ref[...]


@jax.jit
def _tc_final(p2, degp, b1s, batchb, wm0, bm0, wm1, bm1, wm2, bm2,
              wm3p, bm3p):
    fixed = lambda b, r: (0, 0)
    return pl.pallas_call(
        _tc3_body,
        grid=(2, NRT),
        in_specs=[
            pl.BlockSpec((1, RT, D), lambda b, r: (b, r, 0)),
            pl.BlockSpec((1, RT, D), lambda b, r: (b, r, 0)),
            pl.BlockSpec((1, 1, D), lambda b, r: (b, 0, 0)),
            pl.BlockSpec((1, RT, 16), lambda b, r: (b, r, 0)),
            pl.BlockSpec((2 * D, HID), fixed),
            pl.BlockSpec((1, HID), fixed),
            pl.BlockSpec((HID, HID), fixed),
            pl.BlockSpec((1, HID), fixed),
            pl.BlockSpec((HID, HID), fixed),
            pl.BlockSpec((1, HID), fixed),
            pl.BlockSpec((HID, HID), fixed),
            pl.BlockSpec((1, HID), fixed),
        ],
        out_specs=pl.BlockSpec((G, HID), fixed),
        out_shape=jax.ShapeDtypeStruct((G, HID), jnp.float32),
        scratch_shapes=[
            pltpu.VMEM((2, G, D), jnp.float32),
            pltpu.VMEM((2, G, D), jnp.float32),
        ],
    )(p2, degp, b1s, batchb, wm0, bm0, wm1, bm1, wm2, bm2, wm3p, bm3p)


def kernel(x1, edge_index1, x2, edge_index2, batch1, batch2,
           W1_0, b1_0, W1_1, b1_1, W2_0, b2_0, W2_1, b2_1,
           Wm0, bm0, Wm1, bm1, Wm2, bm2, Wm3, bm3):
    src_r = jnp.stack([edge_index1[0], edge_index2[0] + N]).reshape(
        2, NS * N_OUTER, KJ, K)
    dst_r = jnp.stack([edge_index1[1], edge_index2[1]]).reshape(
        2, NS * N_OUTER, KJ, K)
    xs = jnp.stack([x1, x2])
    w0s = jnp.stack([W1_0, W2_0])
    w1s = jnp.stack([W1_1, W2_1])
    b0s = jnp.stack([b1_0, b2_0]).reshape(2, 1, D)
    b1s = jnp.stack([b1_1, b2_1]).reshape(2, 1, D)
    batchb = jnp.broadcast_to(
        jnp.stack([batch1, batch2]).astype(jnp.float32)[:, :, None],
        (2, N, 16))
    wm3p = jnp.pad(Wm3, ((0, 0), (0, HID - Wm3.shape[1])))
    bm3p = jnp.pad(bm3, (0, HID - bm3.shape[0])).reshape(1, HID)
    bm0r = bm0.reshape(1, HID)
    bm1r = bm1.reshape(1, HID)
    bm2r = bm2.reshape(1, HID)

    degp = _sc_degree(dst_r)
    y1 = _tc_first(xs, w0s, degp)
    p1 = _sc_prop(y1.reshape(2 * N, D), src_r, dst_r)
    y2 = _tc_mid(p1, degp, b0s, w1s)
    p2 = _sc_prop(y2.reshape(2 * N, D), src_r, dst_r)
    out = _tc_final(p2, degp, b1s, batchb, Wm0, bm0r, Wm1, bm1r,
                    Wm2, bm2r, wm3p, bm3p)
    return out[:, :Wm3.shape[1]]


# speedup vs baseline: 24.9970x; 1.1516x over previous
import functools

import jax
import jax.numpy as jnp
from jax import lax
from jax.experimental import pallas as pl
from jax.experimental.pallas import tpu as pltpu
from jax.experimental.pallas import tpu_sc as plsc

N = 10000
E = 320000
D = 128
G = 64
HID = 128

NS = 16
ROWS_PT = 624
TAIL_OFF = NS * ROWS_PT
TAIL = N - TAIL_OFF
K = 80
KJ = 50
IDX_LOAD = K * KJ
N_OUTER = E // NS // IDX_LOAD


def _sc_mesh():
    return plsc.VectorSubcoreMesh(core_axis_name="c", subcore_axis_name="s",
                                  num_cores=2, num_subcores=NS)


def _sc_degree_body(dst_hbm, out_hbm, ones_v, didx_v, acc_sh, ssem):
    c = lax.axis_index("c")
    s = lax.axis_index("s")

    def fill(i, _):
        for l in range(D // 16):
            ones_v[i, pl.ds(16 * l, 16)] = jnp.ones((16,), jnp.float32)
        return 0

    lax.fori_loop(0, K, fill, 0)
    for i in range(7):
        pltpu.sync_copy(ones_v,
                        acc_sh.at[pl.ds(s * ROWS_PT + i * K, K)])
    pltpu.sync_copy(ones_v.at[pl.ds(0, ROWS_PT - 7 * K)],
                    acc_sh.at[pl.ds(s * ROWS_PT + 7 * K, ROWS_PT - 7 * K)])

    @pl.when(s == NS - 1)
    def _():
        pltpu.sync_copy(ones_v.at[pl.ds(0, TAIL)],
                        acc_sh.at[pl.ds(TAIL_OFF, TAIL)])

    plsc.subcore_barrier()

    def outer(it, _):
        g = s * N_OUTER + it
        pltpu.sync_copy(dst_hbm.at[c, g], didx_v)
        descs = [pltpu.async_copy(ones_v, acc_sh.at[didx_v.at[j]], ssem,
                                  add=True) for j in range(KJ)]
        for d in descs:
            d.wait()
        return 0

    lax.fori_loop(0, N_OUTER, outer, 0)
    plsc.subcore_barrier()
    pltpu.sync_copy(acc_sh.at[pl.ds(s * ROWS_PT, ROWS_PT)],
                    out_hbm.at[c, pl.ds(s * ROWS_PT, ROWS_PT)])

    @pl.when(s == NS - 1)
    def _():
        pltpu.sync_copy(acc_sh.at[pl.ds(TAIL_OFF, TAIL)],
                        out_hbm.at[c, pl.ds(TAIL_OFF, TAIL)])


@jax.jit
def _sc_degree(dst_r):
    return pl.kernel(
        _sc_degree_body,
        out_type=jax.ShapeDtypeStruct((2, N, D), jnp.float32),
        mesh=_sc_mesh(),
        scratch_types=[
            pltpu.VMEM((K, D), jnp.float32),
            pltpu.VMEM((KJ, K), jnp.int32),
            pltpu.VMEM_SHARED((N, D), jnp.float32),
            pltpu.SemaphoreType.DMA,
        ],
    )(dst_r)


def _sc_prop_body(ys_hbm, src_hbm, dst_hbm, out_hbm,
                  sidx_v, didx_v, rows0_v, rows1_v, rows2_v, acc_sh,
                  gsem0, gsem1, gsem2, ssem0, ssem1, ssem2):
    c = lax.axis_index("c")
    s = lax.axis_index("s")

    pltpu.sync_copy(ys_hbm.at[pl.ds(c * N + s * ROWS_PT, ROWS_PT)],
                    acc_sh.at[pl.ds(s * ROWS_PT, ROWS_PT)])

    @pl.when(s == NS - 1)
    def _():
        pltpu.sync_copy(ys_hbm.at[pl.ds(c * N + TAIL_OFF, TAIL)],
                        acc_sh.at[pl.ds(TAIL_OFF, TAIL)])

    plsc.subcore_barrier()

    rows = [rows0_v, rows1_v, rows2_v]
    gsem = [gsem0, gsem1, gsem2]
    ssem = [ssem0, ssem1, ssem2]
    NB = 3

    def outer(it, _):
        g = s * N_OUTER + it
        pltpu.sync_copy(src_hbm.at[c, g], sidx_v)
        pltpu.sync_copy(dst_hbm.at[c, g], didx_v)
        dg = {}
        ds = {}
        for j in range(KJ):
            p = j % NB
            if j >= NB:
                ds[j - NB].wait()
            dg[j] = pltpu.async_copy(ys_hbm.at[sidx_v.at[j]], rows[p],
                                     gsem[p])
            if j >= 1:
                q = (j - 1) % NB
                dg[j - 1].wait()
                ds[j - 1] = pltpu.async_copy(rows[q],
                                             acc_sh.at[didx_v.at[j - 1]],
                                             ssem[q], add=True)
        q = (KJ - 1) % NB
        dg[KJ - 1].wait()
        ds[KJ - 1] = pltpu.async_copy(rows[q],
                                      acc_sh.at[didx_v.at[KJ - 1]],
                                      ssem[q], add=True)
        for j in range(KJ - NB, KJ):
            ds[j].wait()
        return 0

    lax.fori_loop(0, N_OUTER, outer, 0)
    plsc.subcore_barrier()
    pltpu.sync_copy(acc_sh.at[pl.ds(s * ROWS_PT, ROWS_PT)],
                    out_hbm.at[c, pl.ds(s * ROWS_PT, ROWS_PT)])

    @pl.when(s == NS - 1)
    def _():
        pltpu.sync_copy(acc_sh.at[pl.ds(TAIL_OFF, TAIL)],
                        out_hbm.at[c, pl.ds(TAIL_OFF, TAIL)])


@jax.jit
def _sc_prop(ys2d, src_r, dst_r):
    return pl.kernel(
        _sc_prop_body,
        out_type=jax.ShapeDtypeStruct((2, N, D), jnp.float32),
        mesh=_sc_mesh(),
        scratch_types=[
            pltpu.VMEM((KJ, K), jnp.int32),
            pltpu.VMEM((KJ, K), jnp.int32),
            pltpu.VMEM((K, D), jnp.float32),
            pltpu.VMEM((K, D), jnp.float32),
            pltpu.VMEM((K, D), jnp.float32),
            pltpu.VMEM_SHARED((N, D), jnp.float32),
            pltpu.SemaphoreType.DMA,
            pltpu.SemaphoreType.DMA,
            pltpu.SemaphoreType.DMA,
            pltpu.SemaphoreType.DMA,
            pltpu.SemaphoreType.DMA,
            pltpu.SemaphoreType.DMA,
        ],
    )(ys2d, src_r, dst_r)


RT = 1000
NRT = N // RT


def _tc1_body(x_ref, w_ref, d_ref, o_ref):
    dis = lax.rsqrt(d_ref[0, :, 0:1])
    o_ref[0] = jnp.dot(x_ref[0], w_ref[0],
                       preferred_element_type=jnp.float32) * dis


@jax.jit
def _tc_first(xs, ws, degp):
    return pl.pallas_call(
        _tc1_body,
        grid=(2, NRT),
        in_specs=[
            pl.BlockSpec((1, RT, D), lambda b, r: (b, r, 0)),
            pl.BlockSpec((1, D, D), lambda b, r: (b, 0, 0)),
            pl.BlockSpec((1, RT, D), lambda b, r: (b, r, 0)),
        ],
        out_specs=pl.BlockSpec((1, RT, D), lambda b, r: (b, r, 0)),
        out_shape=jax.ShapeDtypeStruct((2, N, D), jnp.float32),
    )(xs, ws, degp)


def _tc2_body(p_ref, d_ref, b_ref, w_ref, o_ref):
    dis = lax.rsqrt(d_ref[0, :, 0:1])
    h = jnp.maximum(p_ref[0] * dis + b_ref[0], 0.0)
    o_ref[0] = jnp.dot(h, w_ref[0], preferred_element_type=jnp.float32) * dis


@jax.jit
def _tc_mid(p1, degp, b0s, w1s):
    return pl.pallas_call(
        _tc2_body,
        grid=(2, NRT),
        in_specs=[
            pl.BlockSpec((1, RT, D), lambda b, r: (b, r, 0)),
            pl.BlockSpec((1, RT, D), lambda b, r: (b, r, 0)),
            pl.BlockSpec((1, 1, D), lambda b, r: (b, 0, 0)),
            pl.BlockSpec((1, D, D), lambda b, r: (b, 0, 0)),
        ],
        out_specs=pl.BlockSpec((1, RT, D), lambda b, r: (b, r, 0)),
        out_shape=jax.ShapeDtypeStruct((2, N, D), jnp.float32),
    )(p1, degp, b0s, w1s)


def _tc3_body(p_ref, d_ref, b_ref, bt_ref, wm0_ref, bm0_ref, wm1_ref,
              bm1_ref, wm2_ref, bm2_ref, wm3_ref, bm3_ref, o_ref,
              sums, cnts):
    b = pl.program_id(0)
    r = pl.program_id(1)

    @pl.when((b == 0) & (r == 0))
    def _():
        sums[...] = jnp.zeros((2, G, D), jnp.float32)
        cnts[...] = jnp.zeros((2, G, D), jnp.float32)

    dis = lax.rsqrt(d_ref[0, :, 0:1])
    h = jnp.maximum(p_ref[0] * dis + b_ref[0], 0.0)
    bt = bt_ref[0, :, 0:1]
    iota = lax.broadcasted_iota(jnp.int32, (RT, G), 1).astype(jnp.float32)
    oh = (bt == iota).astype(jnp.float32)
    dn = (((0,), (0,)), ((), ()))
    psum = lax.dot_general(oh, h, dn, preferred_element_type=jnp.float32)
    pcnt = lax.dot_general(oh, jnp.ones((RT, D), jnp.float32), dn,
                           preferred_element_type=jnp.float32)
    sums[pl.ds(b, 1)] += psum[None]
    cnts[pl.ds(b, 1)] += pcnt[None]

    @pl.when((b == 1) & (r == NRT - 1))
    def _():
        m0 = sums[0] / jnp.maximum(cnts[0], 1.0)
        m1 = sums[1] / jnp.maximum(cnts[1], 1.0)
        z = jnp.concatenate([m0, m1], axis=1)
        z = jnp.maximum(jnp.dot(z, wm0_ref[...],
                                preferred_element_type=jnp.float32)
                        + bm0_ref[...], 0.0)
        z = jnp.maximum(jnp.dot(z, wm1_ref[...],
                                preferred_element_type=jnp.float32)
                        + bm1_ref[...], 0.0)
        z = jnp.maximum(jnp.dot(z, wm2_ref[...],
                                preferred_element_type=jnp.float32)
                        + bm2_ref[...], 0.0)
        o_ref[...] = jnp.dot(z, wm3_ref[...],
                             preferred_element_type=jnp.float32) + bm3_ref[...]


@jax.jit
def _tc_final(p2, degp, b1s, batchb, wm0, bm0, wm1, bm1, wm2, bm2,
              wm3p, bm3p):
    fixed = lambda b, r: (0, 0)
    return pl.pallas_call(
        _tc3_body,
        grid=(2, NRT),
        in_specs=[
            pl.BlockSpec((1, RT, D), lambda b, r: (b, r, 0)),
            pl.BlockSpec((1, RT, D), lambda b, r: (b, r, 0)),
            pl.BlockSpec((1, 1, D), lambda b, r: (b, 0, 0)),
            pl.BlockSpec((1, RT, 16), lambda b, r: (b, r, 0)),
            pl.BlockSpec((2 * D, HID), fixed),
            pl.BlockSpec((1, HID), fixed),
            pl.BlockSpec((HID, HID), fixed),
            pl.BlockSpec((1, HID), fixed),
            pl.BlockSpec((HID, HID), fixed),
            pl.BlockSpec((1, HID), fixed),
            pl.BlockSpec((HID, HID), fixed),
            pl.BlockSpec((1, HID), fixed),
        ],
        out_specs=pl.BlockSpec((G, HID), fixed),
        out_shape=jax.ShapeDtypeStruct((G, HID), jnp.float32),
        scratch_shapes=[
            pltpu.VMEM((2, G, D), jnp.float32),
            pltpu.VMEM((2, G, D), jnp.float32),
        ],
    )(p2, degp, b1s, batchb, wm0, bm0, wm1, bm1, wm2, bm2, wm3p, bm3p)


def kernel(x1, edge_index1, x2, edge_index2, batch1, batch2,
           W1_0, b1_0, W1_1, b1_1, W2_0, b2_0, W2_1, b2_1,
           Wm0, bm0, Wm1, bm1, Wm2, bm2, Wm3, bm3):
    src_r = jnp.stack([edge_index1[0], edge_index2[0] + N]).reshape(
        2, NS * N_OUTER, KJ, K)
    dst_r = jnp.stack([edge_index1[1], edge_index2[1]]).reshape(
        2, NS * N_OUTER, KJ, K)
    xs = jnp.stack([x1, x2])
    w0s = jnp.stack([W1_0, W2_0])
    w1s = jnp.stack([W1_1, W2_1])
    b0s = jnp.stack([b1_0, b2_0]).reshape(2, 1, D)
    b1s = jnp.stack([b1_1, b2_1]).reshape(2, 1, D)
    batchb = jnp.broadcast_to(
        jnp.stack([batch1, batch2]).astype(jnp.float32)[:, :, None],
        (2, N, 16))
    wm3p = jnp.pad(Wm3, ((0, 0), (0, HID - Wm3.shape[1])))
    bm3p = jnp.pad(bm3, (0, HID - bm3.shape[0])).reshape(1, HID)
    bm0r = bm0.reshape(1, HID)
    bm1r = bm1.reshape(1, HID)
    bm2r = bm2.reshape(1, HID)

    degp = _sc_degree(dst_r)
    y1 = _tc_first(xs, w0s, degp)
    p1 = _sc_prop(y1.reshape(2 * N, D), src_r, dst_r)
    y2 = _tc_mid(p1, degp, b0s, w1s)
    p2 = _sc_prop(y2.reshape(2 * N, D), src_r, dst_r)
    out = _tc_final(p2, degp, b1s, batchb, Wm0, bm0r, Wm1, bm1r,
                    Wm2, bm2r, wm3p, bm3p)
    return out[:, :Wm3.shape[1]]


# speedup vs baseline: 25.0342x; 1.0015x over previous
import functools

import jax
import jax.numpy as jnp
from jax import lax
from jax.experimental import pallas as pl
from jax.experimental.pallas import tpu as pltpu
from jax.experimental.pallas import tpu_sc as plsc

N = 10000
E = 320000
D = 128
G = 64
HID = 128

NS = 16
ROWS_PT = 624
TAIL_OFF = NS * ROWS_PT
TAIL = N - TAIL_OFF
K = 80
KJ = 50
IDX_LOAD = K * KJ
N_OUTER = E // NS // IDX_LOAD


def _sc_mesh():
    return plsc.VectorSubcoreMesh(core_axis_name="c", subcore_axis_name="s",
                                  num_cores=2, num_subcores=NS)


def _sc_degree_body(dst_hbm, out_hbm, ones_v, didx_v, acc_sh, ssem):
    c = lax.axis_index("c")
    s = lax.axis_index("s")

    def fill(i, _):
        for l in range(D // 16):
            ones_v[i, pl.ds(16 * l, 16)] = jnp.ones((16,), jnp.float32)
        return 0

    lax.fori_loop(0, K, fill, 0)
    for i in range(7):
        pltpu.sync_copy(ones_v,
                        acc_sh.at[pl.ds(s * ROWS_PT + i * K, K)])
    pltpu.sync_copy(ones_v.at[pl.ds(0, ROWS_PT - 7 * K)],
                    acc_sh.at[pl.ds(s * ROWS_PT + 7 * K, ROWS_PT - 7 * K)])

    @pl.when(s == NS - 1)
    def _():
        pltpu.sync_copy(ones_v.at[pl.ds(0, TAIL)],
                        acc_sh.at[pl.ds(TAIL_OFF, TAIL)])

    plsc.subcore_barrier()

    def outer(it, _):
        g = s * N_OUTER + it
        pltpu.sync_copy(dst_hbm.at[c, g], didx_v)
        descs = [pltpu.async_copy(ones_v, acc_sh.at[didx_v.at[j]], ssem,
                                  add=True) for j in range(KJ)]
        for d in descs:
            d.wait()
        return 0

    lax.fori_loop(0, N_OUTER, outer, 0)
    plsc.subcore_barrier()
    pltpu.sync_copy(acc_sh.at[pl.ds(s * ROWS_PT, ROWS_PT)],
                    out_hbm.at[c, pl.ds(s * ROWS_PT, ROWS_PT)])

    @pl.when(s == NS - 1)
    def _():
        pltpu.sync_copy(acc_sh.at[pl.ds(TAIL_OFF, TAIL)],
                        out_hbm.at[c, pl.ds(TAIL_OFF, TAIL)])


@jax.jit
def _sc_degree(dst_r):
    return pl.kernel(
        _sc_degree_body,
        out_type=jax.ShapeDtypeStruct((2, N, D), jnp.float32),
        mesh=_sc_mesh(),
        scratch_types=[
            pltpu.VMEM((K, D), jnp.float32),
            pltpu.VMEM((KJ, K), jnp.int32),
            pltpu.VMEM_SHARED((N, D), jnp.float32),
            pltpu.SemaphoreType.DMA,
        ],
    )(dst_r)


def _sc_prop_body(ys_hbm, src_hbm, dst_hbm, out_hbm,
                  sidx_v, didx_v, rows0_v, rows1_v, rows2_v, acc_sh,
                  gsem0, gsem1, gsem2, ssem0, ssem1, ssem2):
    c = lax.axis_index("c")
    s = lax.axis_index("s")

    pltpu.sync_copy(ys_hbm.at[pl.ds(c * N + s * ROWS_PT, ROWS_PT)],
                    acc_sh.at[pl.ds(s * ROWS_PT, ROWS_PT)])

    @pl.when(s == NS - 1)
    def _():
        pltpu.sync_copy(ys_hbm.at[pl.ds(c * N + TAIL_OFF, TAIL)],
                        acc_sh.at[pl.ds(TAIL_OFF, TAIL)])

    plsc.subcore_barrier()

    rows = [rows0_v, rows1_v, rows2_v]
    gsem = [gsem0, gsem1, gsem2]
    ssem = [ssem0, ssem1, ssem2]
    NB = 3

    def outer(it, _):
        g = s * N_OUTER + it
        pltpu.sync_copy(src_hbm.at[c, g], sidx_v)
        pltpu.sync_copy(dst_hbm.at[c, g], didx_v)
        dg = {}
        ds = {}
        for j in range(KJ):
            p = j % NB
            if j >= NB:
                ds[j - NB].wait()
            dg[j] = pltpu.async_copy(ys_hbm.at[sidx_v.at[j]], rows[p],
                                     gsem[p])
            if j >= 1:
                q = (j - 1) % NB
                dg[j - 1].wait()
                ds[j - 1] = pltpu.async_copy(rows[q],
                                             acc_sh.at[didx_v.at[j - 1]],
                                             ssem[q], add=True)
        q = (KJ - 1) % NB
        dg[KJ - 1].wait()
        ds[KJ - 1] = pltpu.async_copy(rows[q],
                                      acc_sh.at[didx_v.at[KJ - 1]],
                                      ssem[q], add=True)
        for j in range(KJ - NB, KJ):
            ds[j].wait()
        return 0

    lax.fori_loop(0, N_OUTER, outer, 0)
    plsc.subcore_barrier()
    pltpu.sync_copy(acc_sh.at[pl.ds(s * ROWS_PT, ROWS_PT)],
                    out_hbm.at[c, pl.ds(s * ROWS_PT, ROWS_PT)])

    @pl.when(s == NS - 1)
    def _():
        pltpu.sync_copy(acc_sh.at[pl.ds(TAIL_OFF, TAIL)],
                        out_hbm.at[c, pl.ds(TAIL_OFF, TAIL)])


@jax.jit
def _sc_prop(ys2d, src_r, dst_r):
    return pl.kernel(
        _sc_prop_body,
        out_type=jax.ShapeDtypeStruct((2, N, D), jnp.float32),
        mesh=_sc_mesh(),
        scratch_types=[
            pltpu.VMEM((KJ, K), jnp.int32),
            pltpu.VMEM((KJ, K), jnp.int32),
            pltpu.VMEM((K, D), jnp.float32),
            pltpu.VMEM((K, D), jnp.float32),
            pltpu.VMEM((K, D), jnp.float32),
            pltpu.VMEM_SHARED((N, D), jnp.float32),
            pltpu.SemaphoreType.DMA,
            pltpu.SemaphoreType.DMA,
            pltpu.SemaphoreType.DMA,
            pltpu.SemaphoreType.DMA,
            pltpu.SemaphoreType.DMA,
            pltpu.SemaphoreType.DMA,
        ],
    )(ys2d, src_r, dst_r)


RT = 1000
NRT = N // RT


def _tc1_body(x_ref, w_ref, d_ref, o_ref):
    dis = lax.rsqrt(d_ref[0, :, 0:1].astype(jnp.float32))
    o_ref[0] = jnp.dot(x_ref[0], w_ref[0],
                       preferred_element_type=jnp.float32) * dis


@jax.jit
def _tc_first(xs, ws, degp):
    return pl.pallas_call(
        _tc1_body,
        grid=(2, NRT),
        in_specs=[
            pl.BlockSpec((1, RT, D), lambda b, r: (b, r, 0)),
            pl.BlockSpec((1, D, D), lambda b, r: (b, 0, 0)),
            pl.BlockSpec((1, RT, D), lambda b, r: (b, r, 0)),
        ],
        out_specs=pl.BlockSpec((1, RT, D), lambda b, r: (b, r, 0)),
        out_shape=jax.ShapeDtypeStruct((2, N, D), jnp.float32),
    )(xs, ws, degp)


def _tc2_body(p_ref, d_ref, b_ref, w_ref, o_ref):
    dis = lax.rsqrt(d_ref[0, :, 0:1].astype(jnp.float32))
    h = jnp.maximum(p_ref[0] * dis + b_ref[0], 0.0)
    o_ref[0] = jnp.dot(h, w_ref[0], preferred_element_type=jnp.float32) * dis


@jax.jit
def _tc_mid(p1, degp, b0s, w1s):
    return pl.pallas_call(
        _tc2_body,
        grid=(2, NRT),
        in_specs=[
            pl.BlockSpec((1, RT, D), lambda b, r: (b, r, 0)),
            pl.BlockSpec((1, RT, D), lambda b, r: (b, r, 0)),
            pl.BlockSpec((1, 1, D), lambda b, r: (b, 0, 0)),
            pl.BlockSpec((1, D, D), lambda b, r: (b, 0, 0)),
        ],
        out_specs=pl.BlockSpec((1, RT, D), lambda b, r: (b, r, 0)),
        out_shape=jax.ShapeDtypeStruct((2, N, D), jnp.float32),
    )(p1, degp, b0s, w1s)


def _tc3_body(p_ref, d_ref, b_ref, bt_ref, wm0_ref, bm0_ref, wm1_ref,
              bm1_ref, wm2_ref, bm2_ref, wm3_ref, bm3_ref, o_ref,
              sums, cnts):
    b = pl.program_id(0)
    r = pl.program_id(1)

    @pl.when((b == 0) & (r == 0))
    def _():
        sums[...] = jnp.zeros((2, G, D), jnp.float32)
        cnts[...] = jnp.zeros((2, G, D), jnp.float32)

    dis = lax.rsqrt(d_ref[0, :, 0:1].astype(jnp.float32))
    h = jnp.maximum(p_ref[0] * dis + b_ref[0], 0.0)
    bt = bt_ref[0, :, 0:1]
    iota = lax.broadcasted_iota(jnp.int32, (RT, G), 1).astype(jnp.float32)
    oh = (bt == iota).astype(jnp.float32)
    dn = (((0,), (0,)), ((), ()))
    psum = lax.dot_general(oh, h, dn, preferred_element_type=jnp.float32)
    pcnt = lax.dot_general(oh, jnp.ones((RT, D), jnp.float32), dn,
                           preferred_element_type=jnp.float32)
    sums[pl.ds(b, 1)] += psum[None]
    cnts[pl.ds(b, 1)] += pcnt[None]

    @pl.when((b == 1) & (r == NRT - 1))
    def _():
        m0 = sums[0] / jnp.maximum(cnts[0], 1.0)
        m1 = sums[1] / jnp.maximum(cnts[1], 1.0)
        z = jnp.concatenate([m0, m1], axis=1)
        z = jnp.maximum(jnp.dot(z, wm0_ref[...],
                                preferred_element_type=jnp.float32)
                        + bm0_ref[...], 0.0)
        z = jnp.maximum(jnp.dot(z, wm1_ref[...],
                                preferred_element_type=jnp.float32)
                        + bm1_ref[...], 0.0)
        z = jnp.maximum(jnp.dot(z, wm2_ref[...],
                                preferred_element_type=jnp.float32)
                        + bm2_ref[...], 0.0)
        o_ref[...] = jnp.dot(z, wm3_ref[...],
                             preferred_element_type=jnp.float32) + bm3_ref[...]


@jax.jit
def _tc_final(p2, degp, b1s, batchb, wm0, bm0, wm1, bm1, wm2, bm2,
              wm3p, bm3p):
    fixed = lambda b, r: (0, 0)
    return pl.pallas_call(
        _tc3_body,
        grid=(2, NRT),
        in_specs=[
            pl.BlockSpec((1, RT, D), lambda b, r: (b, r, 0)),
            pl.BlockSpec((1, RT, D), lambda b, r: (b, r, 0)),
            pl.BlockSpec((1, 1, D), lambda b, r: (b, 0, 0)),
            pl.BlockSpec((1, RT, 16), lambda b, r: (b, r, 0)),
            pl.BlockSpec((2 * D, HID), fixed),
            pl.BlockSpec((1, HID), fixed),
            pl.BlockSpec((HID, HID), fixed),
            pl.BlockSpec((1, HID), fixed),
            pl.BlockSpec((HID, HID), fixed),
            pl.BlockSpec((1, HID), fixed),
            pl.BlockSpec((HID, HID), fixed),
            pl.BlockSpec((1, HID), fixed),
        ],
        out_specs=pl.BlockSpec((G, HID), fixed),
        out_shape=jax.ShapeDtypeStruct((G, HID), jnp.float32),
        scratch_shapes=[
            pltpu.VMEM((2, G, D), jnp.float32),
            pltpu.VMEM((2, G, D), jnp.float32),
        ],
    )(p2, degp, b1s, batchb, wm0, bm0, wm1, bm1, wm2, bm2, wm3p, bm3p)


def kernel(x1, edge_index1, x2, edge_index2, batch1, batch2,
           W1_0, b1_0, W1_1, b1_1, W2_0, b2_0, W2_1, b2_1,
           Wm0, bm0, Wm1, bm1, Wm2, bm2, Wm3, bm3):
    src_r = jnp.stack([edge_index1[0], edge_index2[0] + N]).reshape(
        2, NS * N_OUTER, KJ, K)
    dst_r = jnp.stack([edge_index1[1], edge_index2[1]]).reshape(
        2, NS * N_OUTER, KJ, K)
    xs = jnp.stack([x1, x2])
    w0s = jnp.stack([W1_0, W2_0])
    w1s = jnp.stack([W1_1, W2_1])
    b0s = jnp.stack([b1_0, b2_0]).reshape(2, 1, D)
    b1s = jnp.stack([b1_1, b2_1]).reshape(2, 1, D)
    batchb = jnp.broadcast_to(
        jnp.stack([batch1, batch2]).astype(jnp.float32)[:, :, None],
        (2, N, 16))
    wm3p = jnp.pad(Wm3, ((0, 0), (0, HID - Wm3.shape[1])))
    bm3p = jnp.pad(bm3, (0, HID - bm3.shape[0])).reshape(1, HID)
    bm0r = bm0.reshape(1, HID)
    bm1r = bm1.reshape(1, HID)
    bm2r = bm2.reshape(1, HID)

    degp = _sc_degree(dst_r)
    y1 = _tc_first(xs, w0s, degp)
    p1 = _sc_prop(y1.reshape(2 * N, D), src_r, dst_r)
    y2 = _tc_mid(p1, degp, b0s, w1s)
    p2 = _sc_prop(y2.reshape(2 * N, D), src_r, dst_r)
    out = _tc_final(p2, degp, b1s, batchb, Wm0, bm0r, Wm1, bm1r,
                    Wm2, bm2r, wm3p, bm3p)
    return out[:, :Wm3.shape[1]]
